# baseline (speedup 1.0000x reference)
import functools

import jax
import jax.numpy as jnp
from jax import lax
from jax.experimental import pallas as pl
from jax.experimental.pallas import tpu as pltpu
from jax.experimental.pallas import tpu_sc as plsc

NUM_E = 8
HID = 2048
LANES = 16


def _matmul_t(x_ref, w_ref):
    return lax.dot_general(
        w_ref[...].astype(jnp.bfloat16), x_ref[...].astype(jnp.bfloat16),
        (((1,), (1,)), ((), ())),
        preferred_element_type=jnp.float32,
    )


def _logits_body(x_ref, w_ref, out_ref):
    out_ref[...] = _matmul_t(x_ref, w_ref).reshape(out_ref.shape)


def _tc_logits(xf, w, block_m):
    t = xf.shape[0]
    s = 8192
    return pl.pallas_call(
        _logits_body,
        grid=(t // block_m,),
        in_specs=[
            pl.BlockSpec((block_m, HID), lambda i: (i, 0)),
            pl.BlockSpec((NUM_E, HID), lambda i: (0, 0)),
        ],
        out_specs=pl.BlockSpec(
            (1, NUM_E, block_m),
            lambda i, bm=block_m, s_=s: (i // (s // bm), 0, i % (s // bm)),
        ),
        out_shape=jax.ShapeDtypeStruct((t // s, NUM_E, s), jnp.float32),
    )(xf, w)


def _routed_body(x_ref, w_ref, lg_ref, pb_ref, tp_ref, ti_ref):
    lt = _matmul_t(x_ref, w_ref)
    bm = lt.shape[1]
    lg_ref[...] = lt.reshape(lg_ref.shape)
    m = jnp.max(lt, axis=0, keepdims=True)
    es = jnp.exp(lt - m)
    r = 1.0 / jnp.sum(es, axis=0, keepdims=True)
    pb_ref[...] = (es * r).reshape(pb_ref.shape)
    ls = [lax.slice_in_dim(lt, e, e + 1, axis=0) for e in range(NUM_E)]
    zero = jnp.zeros((1, bm), jnp.int32)
    m1, i1 = ls[0], zero
    m2 = jnp.full((1, bm), -jnp.inf, jnp.float32)
    i2 = zero
    for e in range(1, NUM_E):
        ev = jnp.full((1, bm), e, jnp.int32)
        gt1 = ls[e] > m1
        gt2 = ls[e] > m2
        m2 = jnp.where(gt1, m1, jnp.where(gt2, ls[e], m2))
        i2 = jnp.where(gt1, i1, jnp.where(gt2, ev, i2))
        m1 = jnp.where(gt1, ls[e], m1)
        i1 = jnp.where(gt1, ev, i1)
    tp = jnp.concatenate([jnp.exp(m1 - m) * r, jnp.exp(m2 - m) * r], axis=0)
    ti = jnp.concatenate([i1, i2], axis=0)
    tp_ref[...] = tp.reshape(tp_ref.shape)
    ti_ref[...] = ti.reshape(ti_ref.shape)


def _tc_routed(xf, w, block_m):
    t = xf.shape[0]
    s = 8192
    nb = t // s
    grid = (t // block_m,)
    omap = lambda i, bm=block_m, s_=s: (i // (s_ // bm), 0, i % (s_ // bm))
    return pl.pallas_call(
        _routed_body,
        grid=grid,
        in_specs=[
            pl.BlockSpec((block_m, HID), lambda i: (i, 0)),
            pl.BlockSpec((NUM_E, HID), lambda i: (0, 0)),
        ],
        out_specs=[
            pl.BlockSpec((1, NUM_E, block_m), omap),
            pl.BlockSpec((1, NUM_E, block_m), omap),
            pl.BlockSpec((1, 2, block_m), omap),
            pl.BlockSpec((1, 2, block_m), omap),
        ],
        out_shape=[
            jax.ShapeDtypeStruct((nb, NUM_E, s), jnp.float32),
            jax.ShapeDtypeStruct((nb, NUM_E, s), jnp.float32),
            jax.ShapeDtypeStruct((nb, 2, s), jnp.float32),
            jax.ShapeDtypeStruct((nb, 2, s), jnp.int32),
        ],
    )(xf, w)


def _sc_router_body(tpn, nb, logits_hbm, probs_hbm, tp_hbm, ti_hbm,
                    lbuf, pbuf, tpb, tib, sem):
    nc = 2
    wid = lax.axis_index("s") * nc + lax.axis_index("c")
    wpb = 8192 // tpn
    b = wid // wpb
    soff = (wid % wpb) * tpn

    @pl.when(b < nb)
    def _active():
        pltpu.async_copy(
            logits_hbm.at[b, :, pl.ds(soff, tpn)], lbuf, sem).wait()

        @pl.loop(0, tpn // LANES)
        def _(g):
            tb = g * LANES
            ls = [lbuf[e, pl.ds(tb, LANES)] for e in range(NUM_E)]
            m = ls[0]
            for e in range(1, NUM_E):
                m = jnp.maximum(m, ls[e])
            es = [jnp.exp(l - m) for l in ls]
            ssum = es[0]
            for e in range(1, NUM_E):
                ssum = ssum + es[e]
            r = 1.0 / ssum
            for e in range(NUM_E):
                pbuf[e, pl.ds(tb, LANES)] = es[e] * r
            zero = jnp.zeros((LANES,), jnp.int32)
            m1, i1 = ls[0], zero
            m2 = jnp.full((LANES,), -jnp.inf, jnp.float32)
            i2 = zero
            for e in range(1, NUM_E):
                ev = jnp.full((LANES,), e, jnp.int32)
                gt1 = ls[e] > m1
                gt2 = ls[e] > m2
                m2 = jnp.where(gt1, m1, jnp.where(gt2, ls[e], m2))
                i2 = jnp.where(gt1, i1, jnp.where(gt2, ev, i2))
                m1 = jnp.where(gt1, ls[e], m1)
                i1 = jnp.where(gt1, ev, i1)
            tpb[0, pl.ds(tb, LANES)] = jnp.exp(m1 - m) * r
            tpb[1, pl.ds(tb, LANES)] = jnp.exp(m2 - m) * r
            tib[0, pl.ds(tb, LANES)] = i1
            tib[1, pl.ds(tb, LANES)] = i2

        copies = [
            pltpu.async_copy(pbuf, probs_hbm.at[b, :, pl.ds(soff, tpn)], sem),
            pltpu.async_copy(tpb, tp_hbm.at[b, :, pl.ds(soff, tpn)], sem),
            pltpu.async_copy(tib, ti_hbm.at[b, :, pl.ds(soff, tpn)], sem),
        ]
        for c in copies:
            c.wait()


def _sc_router(logits_t):
    nb, _, s = logits_t.shape
    tpn = 1024
    mesh = plsc.VectorSubcoreMesh(core_axis_name="c", subcore_axis_name="s")
    return pl.kernel(
        functools.partial(_sc_router_body, tpn, nb),
        out_type=[
            jax.ShapeDtypeStruct((nb, NUM_E, s), jnp.float32),
            jax.ShapeDtypeStruct((nb, 2, s), jnp.float32),
            jax.ShapeDtypeStruct((nb, 2, s), jnp.int32),
        ],
        mesh=mesh,
        scratch_types=[
            pltpu.VMEM((NUM_E, tpn), jnp.float32),
            pltpu.VMEM((NUM_E, tpn), jnp.float32),
            pltpu.VMEM((2, tpn), jnp.float32),
            pltpu.VMEM((2, tpn), jnp.int32),
            pltpu.SemaphoreType.DMA,
        ],
        compiler_params=pltpu.CompilerParams(needs_layout_passes=False,
                                             skip_device_barrier=True),
    )(logits_t)


def kernel(x, W):
    b, s, d = x.shape
    nb_sc = b - 1
    xa = x[:nb_sc].reshape(nb_sc * s, d)
    xb = x[nb_sc:].reshape(s, d)
    logits_a = _tc_logits(xa, W, block_m=1024)
    probs_a, tp_a, ti_a = _sc_router(logits_a)
    logits_b, probs_b, tp_b, ti_b = _tc_routed(xb, W, block_m=1024)
    cat = lambda u, v: jnp.concatenate([u, v], axis=0)
    return (
        jnp.transpose(cat(tp_a, tp_b), (0, 2, 1)),
        jnp.transpose(cat(ti_a, ti_b), (0, 2, 1)),
        jnp.transpose(cat(probs_a, probs_b), (0, 2, 1)),
        jnp.transpose(cat(logits_a, logits_b), (0, 2, 1)),
    )


# speedup vs baseline: 2.5498x; 2.5498x over previous
import functools

import jax
import jax.numpy as jnp
from jax import lax
from jax.experimental import pallas as pl
from jax.experimental.pallas import tpu as pltpu
from jax.experimental.pallas import tpu_sc as plsc

NUM_E = 8
HID = 2048
LANES = 16


def _matmul_t(x_ref, w_ref):
    return lax.dot_general(
        w_ref[...].astype(jnp.bfloat16), x_ref[...].astype(jnp.bfloat16),
        (((1,), (1,)), ((), ())),
        preferred_element_type=jnp.float32,
    )


def _logits_body(x_ref, w_ref, out_ref):
    out_ref[...] = _matmul_t(x_ref, w_ref).reshape(out_ref.shape)


def _tc_logits(xf, w, block_m, nb, blk0):
    s = 8192
    return pl.pallas_call(
        _logits_body,
        grid=(nb * s // block_m,),
        in_specs=[
            pl.BlockSpec((block_m, HID), lambda i, b0=blk0: (i + b0, 0)),
            pl.BlockSpec((NUM_E, HID), lambda i: (0, 0)),
        ],
        out_specs=pl.BlockSpec(
            (1, NUM_E, block_m),
            lambda i, bm=block_m, s_=s: (i // (s // bm), 0, i % (s // bm)),
        ),
        out_shape=jax.ShapeDtypeStruct((nb, NUM_E, s), jnp.float32),
    )(xf, w)


def _routed_body(x_ref, w_ref, lg_ref, pb_ref, tp_ref, ti_ref):
    lt = _matmul_t(x_ref, w_ref)
    bm = lt.shape[1]
    lg_ref[...] = lt.reshape(lg_ref.shape)
    m = jnp.max(lt, axis=0, keepdims=True)
    es = jnp.exp(lt - m)
    r = 1.0 / jnp.sum(es, axis=0, keepdims=True)
    pb_ref[...] = (es * r).reshape(pb_ref.shape)
    ls = [lax.slice_in_dim(lt, e, e + 1, axis=0) for e in range(NUM_E)]
    zero = jnp.zeros((1, bm), jnp.int32)
    m1, i1 = ls[0], zero
    m2 = jnp.full((1, bm), -jnp.inf, jnp.float32)
    i2 = zero
    for e in range(1, NUM_E):
        ev = jnp.full((1, bm), e, jnp.int32)
        gt1 = ls[e] > m1
        gt2 = ls[e] > m2
        m2 = jnp.where(gt1, m1, jnp.where(gt2, ls[e], m2))
        i2 = jnp.where(gt1, i1, jnp.where(gt2, ev, i2))
        m1 = jnp.where(gt1, ls[e], m1)
        i1 = jnp.where(gt1, ev, i1)
    tp = jnp.concatenate([jnp.exp(m1 - m) * r, jnp.exp(m2 - m) * r], axis=0)
    ti = jnp.concatenate([i1, i2], axis=0)
    tp_ref[...] = tp.reshape(tp_ref.shape)
    ti_ref[...] = ti.reshape(ti_ref.shape)


def _tc_routed(xf, w, block_m, nb, blk0):
    s = 8192
    grid = (nb * s // block_m,)
    omap = lambda i, bm=block_m, s_=s: (i // (s_ // bm), 0, i % (s_ // bm))
    return pl.pallas_call(
        _routed_body,
        grid=grid,
        in_specs=[
            pl.BlockSpec((block_m, HID), lambda i, b0=blk0: (i + b0, 0)),
            pl.BlockSpec((NUM_E, HID), lambda i: (0, 0)),
        ],
        out_specs=[
            pl.BlockSpec((1, NUM_E, block_m), omap),
            pl.BlockSpec((1, NUM_E, block_m), omap),
            pl.BlockSpec((1, 2, block_m), omap),
            pl.BlockSpec((1, 2, block_m), omap),
        ],
        out_shape=[
            jax.ShapeDtypeStruct((nb, NUM_E, s), jnp.float32),
            jax.ShapeDtypeStruct((nb, NUM_E, s), jnp.float32),
            jax.ShapeDtypeStruct((nb, 2, s), jnp.float32),
            jax.ShapeDtypeStruct((nb, 2, s), jnp.int32),
        ],
    )(xf, w)


def _sc_router_body(tpn, nb, logits_hbm, probs_hbm, tp_hbm, ti_hbm,
                    lbuf, pbuf, tpb, tib, sem):
    nc = 2
    wid = lax.axis_index("s") * nc + lax.axis_index("c")
    wpb = 8192 // tpn
    b = wid // wpb
    soff = (wid % wpb) * tpn

    @pl.when(b < nb)
    def _active():
        pltpu.async_copy(
            logits_hbm.at[b, :, pl.ds(soff, tpn)], lbuf, sem).wait()

        @pl.loop(0, tpn // LANES)
        def _(g):
            tb = g * LANES
            ls = [lbuf[e, pl.ds(tb, LANES)] for e in range(NUM_E)]
            m = ls[0]
            for e in range(1, NUM_E):
                m = jnp.maximum(m, ls[e])
            es = [jnp.exp(l - m) for l in ls]
            ssum = es[0]
            for e in range(1, NUM_E):
                ssum = ssum + es[e]
            r = 1.0 / ssum
            for e in range(NUM_E):
                pbuf[e, pl.ds(tb, LANES)] = es[e] * r
            zero = jnp.zeros((LANES,), jnp.int32)
            m1, i1 = ls[0], zero
            m2 = jnp.full((LANES,), -jnp.inf, jnp.float32)
            i2 = zero
            for e in range(1, NUM_E):
                ev = jnp.full((LANES,), e, jnp.int32)
                gt1 = ls[e] > m1
                gt2 = ls[e] > m2
                m2 = jnp.where(gt1, m1, jnp.where(gt2, ls[e], m2))
                i2 = jnp.where(gt1, i1, jnp.where(gt2, ev, i2))
                m1 = jnp.where(gt1, ls[e], m1)
                i1 = jnp.where(gt1, ev, i1)
            tpb[0, pl.ds(tb, LANES)] = jnp.exp(m1 - m) * r
            tpb[1, pl.ds(tb, LANES)] = jnp.exp(m2 - m) * r
            tib[0, pl.ds(tb, LANES)] = i1
            tib[1, pl.ds(tb, LANES)] = i2

        copies = [
            pltpu.async_copy(pbuf, probs_hbm.at[b, :, pl.ds(soff, tpn)], sem),
            pltpu.async_copy(tpb, tp_hbm.at[b, :, pl.ds(soff, tpn)], sem),
            pltpu.async_copy(tib, ti_hbm.at[b, :, pl.ds(soff, tpn)], sem),
        ]
        for c in copies:
            c.wait()


def _sc_router(logits_t):
    nb, _, s = logits_t.shape
    tpn = 1024
    mesh = plsc.VectorSubcoreMesh(core_axis_name="c", subcore_axis_name="s")
    return pl.kernel(
        functools.partial(_sc_router_body, tpn, nb),
        out_type=[
            jax.ShapeDtypeStruct((nb, NUM_E, s), jnp.float32),
            jax.ShapeDtypeStruct((nb, 2, s), jnp.float32),
            jax.ShapeDtypeStruct((nb, 2, s), jnp.int32),
        ],
        mesh=mesh,
        scratch_types=[
            pltpu.VMEM((NUM_E, tpn), jnp.float32),
            pltpu.VMEM((NUM_E, tpn), jnp.float32),
            pltpu.VMEM((2, tpn), jnp.float32),
            pltpu.VMEM((2, tpn), jnp.int32),
            pltpu.SemaphoreType.DMA,
        ],
        compiler_params=pltpu.CompilerParams(needs_layout_passes=False,
                                             skip_device_barrier=True),
    )(logits_t)


def kernel(x, W):
    b, s, d = x.shape
    nb_sc = b - 1
    bm = 1024
    xf = x.reshape(b * s, d)
    logits_a = _tc_logits(xf, W, bm, nb_sc, 0)
    probs_a, tp_a, ti_a = _sc_router(logits_a)
    logits_b, probs_b, tp_b, ti_b = _tc_routed(
        xf, W, bm, b - nb_sc, nb_sc * s // bm)
    cat = lambda u, v: jnp.concatenate([u, v], axis=0)
    return (
        jnp.transpose(cat(tp_a, tp_b), (0, 2, 1)),
        jnp.transpose(cat(ti_a, ti_b), (0, 2, 1)),
        jnp.transpose(cat(probs_a, probs_b), (0, 2, 1)),
        jnp.transpose(cat(logits_a, logits_b), (0, 2, 1)),
    )


# speedup vs baseline: 2.6580x; 1.0424x over previous
import functools

import jax
import jax.numpy as jnp
from jax import lax
from jax.experimental import pallas as pl
from jax.experimental.pallas import tpu as pltpu
from jax.experimental.pallas import tpu_sc as plsc

NUM_E = 8
HID = 2048
LANES = 16


def _logits_body(x_ref, w_ref, out_ref):
    lt = lax.dot_general(
        w_ref[...].astype(jnp.bfloat16), x_ref[...].astype(jnp.bfloat16),
        (((1,), (1,)), ((), ())),
        preferred_element_type=jnp.float32,
    )
    out_ref[...] = lt.reshape(out_ref.shape)


def _tc_logits(xf, w, block_m):
    t = xf.shape[0]
    s = 8192
    return pl.pallas_call(
        _logits_body,
        grid=(t // block_m,),
        in_specs=[
            pl.BlockSpec((block_m, HID), lambda i: (i, 0)),
            pl.BlockSpec((NUM_E, HID), lambda i: (0, 0)),
        ],
        out_specs=pl.BlockSpec(
            (1, NUM_E, block_m),
            lambda i, bm=block_m, s_=s: (i // (s // bm), 0, i % (s // bm)),
        ),
        out_shape=jax.ShapeDtypeStruct((t // s, NUM_E, s), jnp.float32),
    )(xf, w)


def _sc_router_body(tpn, logits_hbm, probs_hbm, tp_hbm, ti_hbm,
                    lbuf, pbuf, tpb, tib, sem):
    nc = 2
    wid = lax.axis_index("s") * nc + lax.axis_index("c")
    wpb = 8192 // tpn
    b = wid // wpb
    soff = (wid % wpb) * tpn
    pltpu.async_copy(logits_hbm.at[b, :, pl.ds(soff, tpn)], lbuf, sem).wait()

    @pl.loop(0, tpn // LANES)
    def _(g):
        tb = g * LANES
        ls = [lbuf[e, pl.ds(tb, LANES)] for e in range(NUM_E)]
        m = ls[0]
        for e in range(1, NUM_E):
            m = jnp.maximum(m, ls[e])
        es = [jnp.exp(l - m) for l in ls]
        ssum = es[0]
        for e in range(1, NUM_E):
            ssum = ssum + es[e]
        r = 1.0 / ssum
        for e in range(NUM_E):
            pbuf[e, pl.ds(tb, LANES)] = es[e] * r
        zero = jnp.zeros((LANES,), jnp.int32)
        m1, i1 = ls[0], zero
        m2 = jnp.full((LANES,), -jnp.inf, jnp.float32)
        i2 = zero
        for e in range(1, NUM_E):
            ev = jnp.full((LANES,), e, jnp.int32)
            gt1 = ls[e] > m1
            gt2 = ls[e] > m2
            m2 = jnp.where(gt1, m1, jnp.where(gt2, ls[e], m2))
            i2 = jnp.where(gt1, i1, jnp.where(gt2, ev, i2))
            m1 = jnp.where(gt1, ls[e], m1)
            i1 = jnp.where(gt1, ev, i1)
        tpb[0, pl.ds(tb, LANES)] = jnp.exp(m1 - m) * r
        tpb[1, pl.ds(tb, LANES)] = jnp.exp(m2 - m) * r
        tib[0, pl.ds(tb, LANES)] = i1
        tib[1, pl.ds(tb, LANES)] = i2

    copies = [
        pltpu.async_copy(pbuf, probs_hbm.at[b, :, pl.ds(soff, tpn)], sem),
        pltpu.async_copy(tpb, tp_hbm.at[b, :, pl.ds(soff, tpn)], sem),
        pltpu.async_copy(tib, ti_hbm.at[b, :, pl.ds(soff, tpn)], sem),
    ]
    for c in copies:
        c.wait()


def _sc_router(logits_t):
    nb, _, s = logits_t.shape
    tpn = nb * s // 32
    mesh = plsc.VectorSubcoreMesh(core_axis_name="c", subcore_axis_name="s")
    return pl.kernel(
        functools.partial(_sc_router_body, tpn),
        out_type=[
            jax.ShapeDtypeStruct((nb, NUM_E, s), jnp.float32),
            jax.ShapeDtypeStruct((nb, 2, s), jnp.float32),
            jax.ShapeDtypeStruct((nb, 2, s), jnp.int32),
        ],
        mesh=mesh,
        scratch_types=[
            pltpu.VMEM((NUM_E, tpn), jnp.float32),
            pltpu.VMEM((NUM_E, tpn), jnp.float32),
            pltpu.VMEM((2, tpn), jnp.float32),
            pltpu.VMEM((2, tpn), jnp.int32),
            pltpu.SemaphoreType.DMA,
        ],
        compiler_params=pltpu.CompilerParams(needs_layout_passes=False,
                                             skip_device_barrier=True),
    )(logits_t)


def kernel(x, W):
    b, s, d = x.shape
    t = b * s
    xf = x.reshape(t, d)
    logits_t = _tc_logits(xf, W, block_m=1024)
    probs_t, tp_t, ti_t = _sc_router(logits_t)
    return (
        jnp.transpose(tp_t, (0, 2, 1)),
        jnp.transpose(ti_t, (0, 2, 1)),
        jnp.transpose(probs_t, (0, 2, 1)),
        jnp.transpose(logits_t, (0, 2, 1)),
    )
